# baseline (speedup 1.0000x reference)
import functools

import jax
import jax.numpy as jnp
from jax.experimental import pallas as pl

N_NODES = 10000
N_EDGES = 320000
D_NODE = 128
D_EDGE = 16
HID = 128

NODE_BLK = 1000
EDGE_BLK = 2000


def _proj_body(x_ref, wr_ref, wc_ref, pr_ref, pc_ref):
    x = x_ref[...]
    pr_ref[...] = jnp.dot(x, wr_ref[...], preferred_element_type=jnp.float32)
    pc_ref[...] = jnp.dot(x, wc_ref[...], preferred_element_type=jnp.float32)


def _proj(x, wr, wc):
    grid = (N_NODES // NODE_BLK,)
    return pl.pallas_call(
        _proj_body,
        grid=grid,
        in_specs=[
            pl.BlockSpec((NODE_BLK, D_NODE), lambda i: (i, 0)),
            pl.BlockSpec((D_NODE, HID), lambda i: (0, 0)),
            pl.BlockSpec((D_NODE, HID), lambda i: (0, 0)),
        ],
        out_specs=[
            pl.BlockSpec((NODE_BLK, HID), lambda i: (i, 0)),
            pl.BlockSpec((NODE_BLK, HID), lambda i: (i, 0)),
        ],
        out_shape=[
            jax.ShapeDtypeStruct((N_NODES, HID), jnp.float32),
            jax.ShapeDtypeStruct((N_NODES, HID), jnp.float32),
        ],
    )(x, wr, wc)


def _edge_body(gr_ref, gc_ref, ea_ref, we_ref, b0_ref, w1_ref, b1_ref, out_ref):
    pre = gr_ref[...] + gc_ref[...] + jnp.dot(
        ea_ref[...], we_ref[...], preferred_element_type=jnp.float32
    ) + b0_ref[...]
    h = jnp.maximum(pre, 0.0)
    out_ref[...] = jnp.dot(h, w1_ref[...], preferred_element_type=jnp.float32) + b1_ref[...]


def _edge_stage(g_r, g_c, ea, we, b0, w1, b1, n_edges):
    grid = (n_edges // EDGE_BLK,)
    return pl.pallas_call(
        _edge_body,
        grid=grid,
        in_specs=[
            pl.BlockSpec((EDGE_BLK, HID), lambda i: (i, 0)),
            pl.BlockSpec((EDGE_BLK, HID), lambda i: (i, 0)),
            pl.BlockSpec((EDGE_BLK, D_EDGE), lambda i: (i, 0)),
            pl.BlockSpec((D_EDGE, HID), lambda i: (0, 0)),
            pl.BlockSpec((1, HID), lambda i: (0, 0)),
            pl.BlockSpec((HID, D_EDGE), lambda i: (0, 0)),
            pl.BlockSpec((1, D_EDGE), lambda i: (0, 0)),
        ],
        out_specs=pl.BlockSpec((EDGE_BLK, D_EDGE), lambda i: (i, 0)),
        out_shape=jax.ShapeDtypeStruct((n_edges, D_EDGE), jnp.float32),
    )(g_r, g_c, ea, we, b0, w1, b1)


def _node_body(x_ref, p_ref, c_ref, w0x_ref, w0a_ref, b0_ref, w1_ref, b1_ref, out_ref):
    psum = p_ref[0] + p_ref[1]
    cnt = c_ref[0] + c_ref[1]
    agg = psum / jnp.maximum(cnt, 1.0)
    pre = jnp.dot(x_ref[...], w0x_ref[...], preferred_element_type=jnp.float32) + jnp.dot(
        agg, w0a_ref[...], preferred_element_type=jnp.float32
    ) + b0_ref[...]
    h = jnp.maximum(pre, 0.0)
    out_ref[...] = jnp.dot(h, w1_ref[...], preferred_element_type=jnp.float32) + b1_ref[...]


def _node_stage(x, p, c, w0x, w0a, b0, w1, b1):
    grid = (N_NODES // NODE_BLK,)
    return pl.pallas_call(
        _node_body,
        grid=grid,
        in_specs=[
            pl.BlockSpec((NODE_BLK, D_NODE), lambda i: (i, 0)),
            pl.BlockSpec((2, NODE_BLK, D_EDGE), lambda i: (0, i, 0)),
            pl.BlockSpec((2, NODE_BLK, D_EDGE), lambda i: (0, i, 0)),
            pl.BlockSpec((D_NODE, HID), lambda i: (0, 0)),
            pl.BlockSpec((D_EDGE, HID), lambda i: (0, 0)),
            pl.BlockSpec((1, HID), lambda i: (0, 0)),
            pl.BlockSpec((HID, D_NODE), lambda i: (0, 0)),
            pl.BlockSpec((1, D_NODE), lambda i: (0, 0)),
        ],
        out_specs=pl.BlockSpec((NODE_BLK, D_NODE), lambda i: (i, 0)),
        out_shape=jax.ShapeDtypeStruct((N_NODES, D_NODE), jnp.float32),
    )(x, p, c, w0x, w0a, b0, w1, b1)


def _gather_stage(p_r, p_c, row, col):
    return p_r[row], p_c[col]


def _scatter_stage(vals, col):
    s = jax.ops.segment_sum(vals, col, num_segments=N_NODES)
    c = jax.ops.segment_sum(jnp.ones((vals.shape[0], D_EDGE), vals.dtype), col,
                            num_segments=N_NODES)
    return jnp.stack([s, jnp.zeros_like(s)]), jnp.stack([c, jnp.zeros_like(c)])


def _layer(x, ea, row, col, eW0, eb0, eW1, eb1, nW0, nb0, nW1, nb1, counts):
    wr = eW0[:D_NODE]
    wc = eW0[D_NODE:2 * D_NODE]
    we = eW0[2 * D_NODE:]
    p_r, p_c = _proj(x, wr, wc)
    g_r, g_c = _gather_stage(p_r, p_c, row, col)
    eout = _edge_stage(g_r, g_c, ea, we, eb0.reshape(1, -1), eW1, eb1.reshape(1, -1),
                       ea.shape[0])
    psum, cnt = _scatter_stage(eout, col)
    if counts is None:
        counts = cnt
    x_new = _node_stage(x, psum, counts, nW0[:D_NODE], nW0[D_NODE:],
                        nb0.reshape(1, -1), nW1, nb1.reshape(1, -1))
    return x_new, eout, counts


@jax.jit
def kernel(x, edge_attr, l0_eW0, l0_eb0, l0_eW1, l0_eb1, l0_nW0, l0_nb0, l0_nW1,
           l0_nb1, l1_eW0, l1_eb0, l1_eW1, l1_eb1, l1_nW0, l1_nb0, l1_nW1, l1_nb1,
           edge_index):
    row = edge_index[0]
    col = edge_index[1]
    x, ea, counts = _layer(x, edge_attr, row, col, l0_eW0, l0_eb0, l0_eW1, l0_eb1,
                           l0_nW0, l0_nb0, l0_nW1, l0_nb1, None)
    x, _, _ = _layer(x, ea, row, col, l1_eW0, l1_eb0, l1_eW1, l1_eb1,
                     l1_nW0, l1_nb0, l1_nW1, l1_nb1, counts)
    return x


# speedup vs baseline: 1.4591x; 1.4591x over previous
import functools

import jax
import jax.numpy as jnp
from jax import lax
from jax.experimental import pallas as pl
from jax.experimental.pallas import tpu as pltpu
from jax.experimental.pallas import tpu_sc as plsc
from jax._src.pallas import core as _pl_core

N_NODES = 10000
N_EDGES = 320000
D_NODE = 128
D_EDGE = 16
HID = 128

NC = 2
NS = 16
NW = NC * NS

CHUNK = 128
KW = 80
EW = KW * CHUNK
E_PAD = NW * EW
IDX_ROWS = E_PAD // CHUNK
TAB_N = 10240
DUMMY = N_NODES
ZROWS = TAB_N // NS

NODE_BLK = 1000
EDGE_BLK = 4096

_mesh = plsc.VectorSubcoreMesh(core_axis_name="c", subcore_axis_name="s")
_scalar_mesh = plsc.ScalarSubcoreMesh(axis_name="c", num_cores=NC)


def _proj_body(x_ref, wr_ref, wc_ref, pr_ref, pc_ref):
    x = x_ref[...]
    pr_ref[...] = jnp.dot(x, wr_ref[...], preferred_element_type=jnp.float32)
    pc_ref[...] = jnp.dot(x, wc_ref[...], preferred_element_type=jnp.float32)


def _proj(x, wr, wc):
    return pl.pallas_call(
        _proj_body,
        grid=(N_NODES // NODE_BLK,),
        in_specs=[
            pl.BlockSpec((NODE_BLK, D_NODE), lambda i: (i, 0)),
            pl.BlockSpec((D_NODE, HID), lambda i: (0, 0)),
            pl.BlockSpec((D_NODE, HID), lambda i: (0, 0)),
        ],
        out_specs=[
            pl.BlockSpec((NODE_BLK, HID), lambda i: (i, 0)),
            pl.BlockSpec((NODE_BLK, HID), lambda i: (i, 0)),
        ],
        out_shape=[
            jax.ShapeDtypeStruct((TAB_N, HID), jnp.float32),
            jax.ShapeDtypeStruct((TAB_N, HID), jnp.float32),
        ],
    )(x, wr, wc)


@functools.partial(
    pl.kernel,
    out_type=[
        jax.ShapeDtypeStruct((E_PAD, HID), jnp.float32),
        jax.ShapeDtypeStruct((E_PAD, HID), jnp.float32),
    ],
    mesh=_mesh,
    scratch_types=[
        pltpu.VMEM((KW, CHUNK), jnp.int32),
        pltpu.VMEM((KW, CHUNK), jnp.int32),
        pltpu.VMEM((2, CHUNK, HID), jnp.float32),
        pltpu.VMEM((2, CHUNK, HID), jnp.float32),
        pltpu.SemaphoreType.DMA((2,)),
        pltpu.SemaphoreType.DMA((2,)),
        pltpu.SemaphoreType.DMA((2,)),
        pltpu.SemaphoreType.DMA((2,)),
    ],
)
def _gather_sc(pr_h, pc_h, rowi_h, coli_h, gr_h, gc_h,
               rows_v, cols_v, bufa, bufb, sga, sgb, swa, swb):
    wid = lax.axis_index("s") * NC + lax.axis_index("c")
    pltpu.sync_copy(rowi_h.at[pl.ds(wid * KW, KW)], rows_v)
    pltpu.sync_copy(coli_h.at[pl.ds(wid * KW, KW)], cols_v)
    ebase = wid * EW

    def gather(g, b):
        return (
            pltpu.make_async_copy(pr_h.at[rows_v.at[g]], bufa.at[b], sga.at[b]),
            pltpu.make_async_copy(pc_h.at[cols_v.at[g]], bufb.at[b], sgb.at[b]),
        )

    def write(g, b):
        dst = pl.ds(ebase + g * CHUNK, CHUNK)
        return (
            pltpu.make_async_copy(bufa.at[b], gr_h.at[dst], swa.at[b]),
            pltpu.make_async_copy(bufb.at[b], gc_h.at[dst], swb.at[b]),
        )

    for b in range(2):
        for h in gather(b, b):
            h.start()

    @pl.loop(0, KW, step=2)
    def _step(g0):
        for b in range(2):
            g = g0 + b
            for h in gather(g, b):
                h.wait()
            for h in write(g, b):
                h.start()

            @pl.when(g + 2 < KW)
            def _():
                for h in write(g, b):
                    h.wait()
                for h in gather(g + 2, b):
                    h.start()

    for b in range(2):
        for h in write(KW - 2 + b, b):
            h.wait()


def _make_scatter(with_counts):
    out_type = [jax.ShapeDtypeStruct((NC, TAB_N, D_EDGE), jnp.float32)]
    if with_counts:
        out_type = out_type * 2
    acc_types = [pltpu.VMEM_SHARED((TAB_N, D_EDGE), jnp.float32)]
    if with_counts:
        acc_types = acc_types * 2

    def _scs_body(*refs):
        pass

    def _tec_body(vals_h, coli_h, *refs):
        if with_counts:
            psum_h, cnt_h, accs, accc = refs
        else:
            psum_h, accs = refs
            accc = None
        cid = lax.axis_index("c")
        sid = lax.axis_index("s")
        wid = sid * NC + cid

        def inner(cols_v, vbuf, stage, sv):
            pltpu.sync_copy(coli_h.at[pl.ds(wid * KW, KW)], cols_v)
            @pl.loop(0, CHUNK)
            def _z(i):
                stage[i, :] = jnp.zeros((D_EDGE,), jnp.float32)

            for j in range(ZROWS // CHUNK):
                zrows = pl.ds(sid * ZROWS + j * CHUNK, CHUNK)
                pltpu.sync_copy(stage, accs.at[zrows])
                if with_counts:
                    pltpu.sync_copy(stage, accc.at[zrows])
            plsc.subcore_barrier()

            if with_counts:
                @pl.loop(0, CHUNK)
                def _o(i):
                    stage[i, :] = jnp.ones((D_EDGE,), jnp.float32)

            def issue(g, b):
                pltpu.make_async_copy(
                    vals_h.at[pl.ds(wid * EW + g * CHUNK, CHUNK)], vbuf.at[b],
                    sv.at[b]).start()

            def wait(g, b):
                pltpu.make_async_copy(
                    vals_h.at[pl.ds(wid * EW + g * CHUNK, CHUNK)], vbuf.at[b],
                    sv.at[b]).wait()

            issue(0, 0)
            issue(1, 1)

            @pl.loop(0, KW, step=2)
            def _step(g0):
                for b in range(2):
                    g = g0 + b
                    wait(g, b)
                    pltpu.sync_copy(vbuf.at[b], accs.at[cols_v.at[g]], add=True)
                    if with_counts:
                        pltpu.sync_copy(stage.at[pl.ds(0, CHUNK)],
                                        accc.at[cols_v.at[g]], add=True)

                    @pl.when(g + 2 < KW)
                    def _():
                        issue(g + 2, b)

            plsc.subcore_barrier()
            myrows = pl.ds(sid * ZROWS, ZROWS)
            pltpu.sync_copy(accs.at[myrows], psum_h.at[cid, myrows])
            if with_counts:
                pltpu.sync_copy(accc.at[myrows], cnt_h.at[cid, myrows])

        pl.run_scoped(
            inner,
            pltpu.VMEM((KW, CHUNK), jnp.int32),
            pltpu.VMEM((2, CHUNK, D_EDGE), jnp.float32),
            pltpu.VMEM((CHUNK, D_EDGE), jnp.float32),
            pltpu.SemaphoreType.DMA((2,)),
        )

    return pl.kernel(
        body=[_scs_body, _tec_body],
        mesh=[_scalar_mesh, _mesh],
        out_type=out_type,
        scratch_types=acc_types,
    )


_scatter_with_counts = _make_scatter(True)
_scatter_no_counts = _make_scatter(False)


def _edge_body(gr_ref, gc_ref, ea_ref, we_ref, b0_ref, w1_ref, b1_ref, out_ref):
    pre = gr_ref[...] + gc_ref[...] + jnp.dot(
        ea_ref[...], we_ref[...], preferred_element_type=jnp.float32
    ) + b0_ref[...]
    h = jnp.maximum(pre, 0.0)
    out_ref[...] = jnp.dot(h, w1_ref[...], preferred_element_type=jnp.float32) + b1_ref[...]


def _edge_stage(g_r, g_c, ea, we, b0, w1, b1):
    return pl.pallas_call(
        _edge_body,
        grid=(E_PAD // EDGE_BLK,),
        in_specs=[
            pl.BlockSpec((EDGE_BLK, HID), lambda i: (i, 0)),
            pl.BlockSpec((EDGE_BLK, HID), lambda i: (i, 0)),
            pl.BlockSpec((EDGE_BLK, D_EDGE), lambda i: (i, 0)),
            pl.BlockSpec((D_EDGE, HID), lambda i: (0, 0)),
            pl.BlockSpec((1, HID), lambda i: (0, 0)),
            pl.BlockSpec((HID, D_EDGE), lambda i: (0, 0)),
            pl.BlockSpec((1, D_EDGE), lambda i: (0, 0)),
        ],
        out_specs=pl.BlockSpec((EDGE_BLK, D_EDGE), lambda i: (i, 0)),
        out_shape=jax.ShapeDtypeStruct((E_PAD, D_EDGE), jnp.float32),
    )(g_r, g_c, ea, we, b0, w1, b1)


def _node_body(x_ref, p_ref, c_ref, w0x_ref, w0a_ref, b0_ref, w1_ref, b1_ref, out_ref):
    psum = p_ref[0] + p_ref[1]
    cnt = c_ref[0] + c_ref[1]
    agg = psum / jnp.maximum(cnt, 1.0)
    pre = jnp.dot(x_ref[...], w0x_ref[...], preferred_element_type=jnp.float32) + jnp.dot(
        agg, w0a_ref[...], preferred_element_type=jnp.float32
    ) + b0_ref[...]
    h = jnp.maximum(pre, 0.0)
    out_ref[...] = jnp.dot(h, w1_ref[...], preferred_element_type=jnp.float32) + b1_ref[...]


def _node_stage(x, p, c, w0x, w0a, b0, w1, b1):
    return pl.pallas_call(
        _node_body,
        grid=(N_NODES // NODE_BLK,),
        in_specs=[
            pl.BlockSpec((NODE_BLK, D_NODE), lambda i: (i, 0)),
            pl.BlockSpec((NC, NODE_BLK, D_EDGE), lambda i: (0, i, 0)),
            pl.BlockSpec((NC, NODE_BLK, D_EDGE), lambda i: (0, i, 0)),
            pl.BlockSpec((D_NODE, HID), lambda i: (0, 0)),
            pl.BlockSpec((D_EDGE, HID), lambda i: (0, 0)),
            pl.BlockSpec((1, HID), lambda i: (0, 0)),
            pl.BlockSpec((HID, D_NODE), lambda i: (0, 0)),
            pl.BlockSpec((1, D_NODE), lambda i: (0, 0)),
        ],
        out_specs=pl.BlockSpec((NODE_BLK, D_NODE), lambda i: (i, 0)),
        out_shape=jax.ShapeDtypeStruct((N_NODES, D_NODE), jnp.float32),
    )(x, p, c, w0x, w0a, b0, w1, b1)


def _layer(x, ea, rowi, coli, eW0, eb0, eW1, eb1, nW0, nb0, nW1, nb1, counts):
    p_r, p_c = _proj(x, eW0[:D_NODE], eW0[D_NODE:2 * D_NODE])
    g_r, g_c = _gather_sc(p_r, p_c, rowi, coli)
    eout = _edge_stage(g_r, g_c, ea, eW0[2 * D_NODE:], eb0.reshape(1, -1),
                       eW1, eb1.reshape(1, -1))
    if True:
        col_flat = coli.reshape(-1)
        s = jax.ops.segment_sum(eout, col_flat, num_segments=TAB_N)
        c = jax.ops.segment_sum(jnp.ones((E_PAD, D_EDGE), jnp.float32),
                                col_flat, num_segments=TAB_N)
        psum = jnp.stack([s, jnp.zeros_like(s)])
        if counts is None:
            counts = jnp.stack([c, jnp.zeros_like(c)])
    elif counts is None:
        psum, counts = _scatter_with_counts(eout, coli)
    else:
        (psum,) = _scatter_no_counts(eout, coli)
    x_new = _node_stage(x, psum, counts, nW0[:D_NODE], nW0[D_NODE:],
                        nb0.reshape(1, -1), nW1, nb1.reshape(1, -1))
    return x_new, eout, counts


@jax.jit
def kernel(x, edge_attr, l0_eW0, l0_eb0, l0_eW1, l0_eb1, l0_nW0, l0_nb0, l0_nW1,
           l0_nb1, l1_eW0, l1_eb0, l1_eW1, l1_eb1, l1_nW0, l1_nb0, l1_nW1, l1_nb1,
           edge_index):
    pad = E_PAD - N_EDGES
    idx = jnp.concatenate(
        [edge_index, jnp.full((2, pad), DUMMY, jnp.int32)], axis=1)
    rowi = idx[0].reshape(IDX_ROWS, CHUNK)
    coli = idx[1].reshape(IDX_ROWS, CHUNK)
    ea = jnp.concatenate(
        [edge_attr, jnp.zeros((pad, D_EDGE), jnp.float32)], axis=0)

    x, ea, counts = _layer(x, ea, rowi, coli, l0_eW0, l0_eb0, l0_eW1, l0_eb1,
                           l0_nW0, l0_nb0, l0_nW1, l0_nb1, None)
    x, _, _ = _layer(x, ea, rowi, coli, l1_eW0, l1_eb0, l1_eW1, l1_eb1,
                     l1_nW0, l1_nb0, l1_nW1, l1_nb1, counts)
    return x
